# Initial kernel scaffold; baseline (speedup 1.0000x reference)
import jax
import jax.numpy as jnp
from jax.experimental import pallas as pl


def kernel(pred_attn, target_attn, batch_target):
    raise NotImplementedError("write your pallas kernel here")


# speedup vs baseline: 5.7290x; 5.7290x over previous
import jax
import jax.numpy as jnp
from jax.experimental import pallas as pl
from jax.experimental.pallas import tpu as pltpu


def _loss_body(pred_ref, tattn_ref, bt_ref, p_ref, m_ref):
    tb = pl.program_id(1)
    ta = tattn_ref[0]
    kdim = ta.shape[0]
    kiota = jax.lax.broadcasted_iota(jnp.int32, ta.shape, 0)
    tmax = jnp.max(ta, axis=0, keepdims=True)
    tgt = jnp.min(jnp.where(ta == tmax, kiota, kdim), axis=0)
    onehot = kiota == tgt[None, :]

    maskf = (bt_ref[0, 0] != -1).astype(jnp.float32)

    acc = jnp.zeros_like(maskf)
    for i in range(pred_ref.shape[0]):
        x = pred_ref[i, 0]
        xm = jnp.max(x, axis=0)
        lse = xm + jnp.log(jnp.sum(jnp.exp(x - xm[None, :]), axis=0))
        picked = jnp.sum(jnp.where(onehot, x, 0.0), axis=0)
        acc = acc + (picked - lse)

    psum = jnp.sum(acc * maskf)
    msum = jnp.sum(maskf)

    @pl.when(tb == 0)
    def _():
        p_ref[...] = jnp.zeros_like(p_ref)
        m_ref[...] = jnp.zeros_like(m_ref)

    p_ref[...] += psum
    m_ref[...] += msum


def kernel(pred_attn, target_attn, batch_target):
    L, B, K, T = pred_attn.shape
    Tb = 256
    bt3 = batch_target.astype(jnp.int32).reshape(B, 1, T)

    grid = (B, T // Tb)
    p, m = pl.pallas_call(
        _loss_body,
        grid=grid,
        in_specs=[
            pl.BlockSpec((L, 1, K, Tb), lambda b, t: (0, b, 0, t)),
            pl.BlockSpec((1, K, Tb), lambda b, t: (b, 0, t)),
            pl.BlockSpec((1, 1, Tb), lambda b, t: (b, 0, t)),
        ],
        out_specs=[
            pl.BlockSpec((1, 1, 128), lambda b, t: (b, 0, 0)),
            pl.BlockSpec((1, 1, 128), lambda b, t: (b, 0, 0)),
        ],
        out_shape=[
            jax.ShapeDtypeStruct((B, 1, 128), jnp.float32),
            jax.ShapeDtypeStruct((B, 1, 128), jnp.float32),
        ],
    )(pred_attn, target_attn, bt3)

    psum = p[:, 0, 0]
    denom = jnp.maximum(m[:, 0, 0], 1.0)
    return -jnp.sum(psum / denom) / (L * B)


# speedup vs baseline: 5.9447x; 1.0377x over previous
import jax
import jax.numpy as jnp
from jax.experimental import pallas as pl
from jax.experimental.pallas import tpu as pltpu


def _loss_body(pred_ref, tattn_ref, bt_ref, p_ref, m_ref):
    tb = pl.program_id(1)
    ta = tattn_ref[0]
    kdim = ta.shape[0]
    kiota = jax.lax.broadcasted_iota(jnp.int32, ta.shape, 0)
    tmax = jnp.max(ta, axis=0, keepdims=True)
    tgt = jnp.min(jnp.where(ta == tmax, kiota, kdim), axis=0)
    onehot = kiota == tgt[None, :]

    maskf = (bt_ref[0, 0] != -1).astype(jnp.float32)

    acc = jnp.zeros_like(maskf)
    for i in range(pred_ref.shape[0]):
        x = pred_ref[i, 0]
        xm = jnp.max(x, axis=0)
        lse = xm + jnp.log(jnp.sum(jnp.exp(x - xm[None, :]), axis=0))
        picked = jnp.sum(jnp.where(onehot, x, 0.0), axis=0)
        acc = acc + (picked - lse)

    psum = jnp.sum(acc * maskf)
    msum = jnp.sum(maskf)

    @pl.when(tb == 0)
    def _():
        p_ref[...] = jnp.zeros_like(p_ref)
        m_ref[...] = jnp.zeros_like(m_ref)

    p_ref[...] += psum
    m_ref[...] += msum


def kernel(pred_attn, target_attn, batch_target):
    L, B, K, T = pred_attn.shape
    Tb = 512
    bt3 = batch_target.astype(jnp.int32).reshape(B, 1, T)

    grid = (B, T // Tb)
    p, m = pl.pallas_call(
        _loss_body,
        grid=grid,
        in_specs=[
            pl.BlockSpec((L, 1, K, Tb), lambda b, t: (0, b, 0, t)),
            pl.BlockSpec((1, K, Tb), lambda b, t: (b, 0, t)),
            pl.BlockSpec((1, 1, Tb), lambda b, t: (b, 0, t)),
        ],
        out_specs=[
            pl.BlockSpec((1, 1, 128), lambda b, t: (b, 0, 0)),
            pl.BlockSpec((1, 1, 128), lambda b, t: (b, 0, 0)),
        ],
        out_shape=[
            jax.ShapeDtypeStruct((B, 1, 128), jnp.float32),
            jax.ShapeDtypeStruct((B, 1, 128), jnp.float32),
        ],
    )(pred_attn, target_attn, bt3)

    psum = p[:, 0, 0]
    denom = jnp.maximum(m[:, 0, 0], 1.0)
    return -jnp.sum(psum / denom) / (L * B)
